# baseline (speedup 1.0000x reference)
import functools

import jax
import jax.numpy as jnp
from jax.experimental import pallas as pl
from jax.experimental.pallas import tpu as pltpu

EPS_ = 1e-6


def _stats_kernel(p_ref, g_ref, label_ref, out_t_ref, out_v_ref,
                  acc_ref, val_ref, *, C):
    b = pl.program_id(0)
    c = pl.program_id(1)
    p = p_ref[0, 0]
    g = g_ref[0, 0]
    pos = jnp.sum(g * p)
    s = jnp.sum(p)
    mx = jnp.max(g)
    ratio = (s - pos) / (pos + EPS_)
    w = jnp.where(label_ref[b] == c, 1.0, 1.0 / C)
    contrib = jnp.where(mx != 0.0, ratio * w, 0.0)

    @pl.when(c == 0)
    def _():
        acc_ref[0] = 0.0
        val_ref[0] = 0

    acc_ref[0] = acc_ref[0] + contrib
    val_ref[0] = val_ref[0] | (mx != 0.0).astype(jnp.int32)

    @pl.when(c == C - 1)
    def _():
        out_t_ref[0, 0, :] = jnp.full((128,), acc_ref[0], jnp.float32)
        out_v_ref[0, 0, :] = jnp.full((128,), val_ref[0], jnp.int32)


def _finalize_kernel(t_ref, v_ref, out_ref):
    total = jnp.sum(t_ref[:, 0, 0:1])
    n_valid = jnp.sum(v_ref[:, 0, 0:1])
    n = jnp.maximum(n_valid, 1).astype(jnp.float32)
    out_ref[0, 0] = jnp.where(total == 0.0, 0.0, jnp.log(total) / n)


@jax.jit
def kernel(Y_pred, Y_gt, label):
    B, C, H, W = Y_pred.shape
    label32 = label.astype(jnp.int32)

    out_t, out_v = pl.pallas_call(
        functools.partial(_stats_kernel, C=C),
        grid=(B, C),
        in_specs=[
            pl.BlockSpec((1, 1, H, W), lambda b, c: (b, c, 0, 0)),
            pl.BlockSpec((1, 1, H, W), lambda b, c: (b, c, 0, 0)),
            pl.BlockSpec(memory_space=pltpu.SMEM),
        ],
        out_specs=[
            pl.BlockSpec((1, 1, 128), lambda b, c: (b, 0, 0)),
            pl.BlockSpec((1, 1, 128), lambda b, c: (b, 0, 0)),
        ],
        out_shape=[
            jax.ShapeDtypeStruct((B, 1, 128), jnp.float32),
            jax.ShapeDtypeStruct((B, 1, 128), jnp.int32),
        ],
        scratch_shapes=[
            pltpu.SMEM((1,), jnp.float32),
            pltpu.SMEM((1,), jnp.int32),
        ],
        compiler_params=pltpu.CompilerParams(
            dimension_semantics=("parallel", "arbitrary"),
        ),
    )(Y_pred, Y_gt, label32)

    out = pl.pallas_call(
        _finalize_kernel,
        out_specs=pl.BlockSpec(memory_space=pltpu.SMEM),
        out_shape=jax.ShapeDtypeStruct((1, 1), jnp.float32),
    )(out_t, out_v)
    return out[0, 0]
